# Initial kernel scaffold; baseline (speedup 1.0000x reference)
import jax
import jax.numpy as jnp
from jax.experimental import pallas as pl


def kernel(x, W1, b1, a1, W2, b2, a2, Wm1, bm1, am, Wm2, bm2, label_edge, mp_edge):
    raise NotImplementedError("write your pallas kernel here")


# speedup vs baseline: 265.5309x; 265.5309x over previous
import functools

import jax
import jax.numpy as jnp
from jax import lax
from jax.experimental import pallas as pl
from jax.experimental.pallas import tpu as pltpu
from jax.experimental.pallas import tpu_sc as plsc

N = 400
BSZ = 8
UT = N * (N - 1) // 2
CONV = 128
DNN = 64
XROWS = 624
UNPACK_CHUNKS = (
    ((0, 56), (56, 48)),
    ((104, 56), (160, 48)),
    ((208, 48), (256, 48)),
    ((304, 48), (352, 48)),
)
UNPACK_BUF_ROWS = 56
PACK_W = 20480
PACK_ROWS = ((0, 56), (48, 128), (120, 208), (200, 400))
PACK_MAX_ROWS = 200


def _iota16():
    return lax.broadcasted_iota(jnp.int32, (16,), 0)


@functools.cache
def _sc_mesh():
    return plsc.VectorSubcoreMesh(
        core_axis_name="c", subcore_axis_name="s", num_cores=2, num_subcores=16
    )


@functools.cache
def _sc_unpack_call():
    @functools.partial(
        pl.kernel,
        out_type=jax.ShapeDtypeStruct((BSZ, N, N), jnp.float32),
        mesh=_sc_mesh(),
        scratch_types=[
            pltpu.VMEM((XROWS * 128,), jnp.float32),
            pltpu.VMEM((UNPACK_BUF_ROWS, N), jnp.float32),
        ],
        compiler_params=pltpu.CompilerParams(needs_layout_passes=False),
    )
    def _sc_unpack(x_hbm, a_hbm, xbuf, obuf):
        wid = lax.axis_index("s") * 2 + lax.axis_index("c")
        b = wid // 4
        q = wid % 4
        pltpu.sync_copy(x_hbm.at[pl.ds(b * (XROWS * 128), XROWS * 128)], xbuf)
        iot = _iota16()

        def variant(qq):
            def _go():
                for r0, cnt in UNPACK_CHUNKS[qq]:
                    def row_body(r, _, r0=r0):
                        i = r0 + r

                        def col_body(c, _):
                            j = c * 16 + iot
                            iv = jnp.broadcast_to(i, (16,))
                            lo = jnp.minimum(iv, j)
                            hi = jnp.maximum(iv, j)
                            idx = (399 * lo - jnp.right_shift(lo * (lo - 1), 1)
                                   + hi - lo - 1)
                            idx = jnp.where(iv == j, UT, idx)
                            vals = plsc.load_gather(xbuf, [idx])
                            obuf[r, pl.ds(c * 16, 16)] = vals
                            return 0

                        lax.fori_loop(0, 25, col_body, 0)
                        return 0

                    lax.fori_loop(0, cnt, row_body, 0)
                    pltpu.sync_copy(
                        obuf.at[pl.ds(0, cnt)], a_hbm.at[b, pl.ds(r0, cnt)]
                    )

            return _go

        for qq in range(4):
            pl.when(q == qq)(variant(qq))

    return _sc_unpack


def _tc_body(a_ref, w1t_ref, w2t_ref, wma_ref, wmb_ref, wm2_ref,
             b1_ref, b2_ref, bm1_ref, scal_ref, y_ref):
    a1 = scal_ref[0, 0]
    a2 = scal_ref[0, 1]
    am = scal_ref[0, 2]
    bm2 = scal_ref[0, 3]

    A = a_ref[0]
    deg = jnp.sum(A, axis=1, keepdims=True)
    degr = jnp.sum(A, axis=0, keepdims=True)
    dc = jnp.where(deg > 0, lax.rsqrt(deg), 0.0)
    dr = jnp.where(degr > 0, lax.rsqrt(degr), 0.0)
    An = A * dc * dr

    t1 = jnp.dot(An, w1t_ref[...], preferred_element_type=jnp.float32) + b1_ref[...]
    h1 = jnp.where(t1 >= 0, t1, a1 * t1)
    t2a = jnp.dot(h1, w2t_ref[...], preferred_element_type=jnp.float32)
    t2 = jnp.dot(An, t2a, preferred_element_type=jnp.float32) + b2_ref[...]
    h2 = jnp.where(t2 >= 0, t2, a2 * t2)

    P = jnp.dot(h2, wma_ref[...], preferred_element_type=jnp.float32) + bm1_ref[...]
    Q = jnp.dot(h2, wmb_ref[...], preferred_element_type=jnp.float32)
    QT = Q.T

    acc = jnp.full((N, N), 0.0, jnp.float32)
    for k in range(DNN):
        t = P[:, k:k + 1] + QT[k:k + 1, :]
        acc = acc + wm2_ref[0, k] * jnp.where(t >= 0, t, am * t)
    y_ref[0] = acc + bm2


def _tc_dense(A, w1t, w2t, wma, wmb, wm2, b1r, b2r, bm1r, scal):
    full = lambda shape: pl.BlockSpec(shape, lambda b: (0,) * len(shape))
    return pl.pallas_call(
        _tc_body,
        grid=(BSZ,),
        in_specs=[
            pl.BlockSpec((1, N, N), lambda b: (b, 0, 0)),
            full((N, CONV)), full((CONV, CONV)), full((CONV, DNN)),
            full((CONV, DNN)), full((1, DNN)), full((1, CONV)),
            full((1, CONV)), full((1, DNN)), full((1, 4)),
        ],
        out_specs=pl.BlockSpec((1, N, N), lambda b: (b, 0, 0)),
        out_shape=jax.ShapeDtypeStruct((BSZ, N, N), jnp.float32),
    )(A, w1t, w2t, wma, wmb, wm2, b1r, b2r, bm1r, scal)


@functools.cache
def _sc_pack_call():
    @functools.partial(
        pl.kernel,
        out_type=jax.ShapeDtypeStruct((BSZ * 4 * PACK_W,), jnp.float32),
        mesh=_sc_mesh(),
        scratch_types=[
            pltpu.VMEM((PACK_MAX_ROWS, N), jnp.float32),
            pltpu.VMEM((PACK_W,), jnp.float32),
        ],
        compiler_params=pltpu.CompilerParams(needs_layout_passes=False),
    )
    def _sc_pack(yf_hbm, out_hbm, ybuf, obuf):
        wid = lax.axis_index("s") * 2 + lax.axis_index("c")
        b = wid // 4
        q = wid % 4
        iot = _iota16()

        def variant(qq):
            rlo, rhi = PACK_ROWS[qq]
            nr = rhi - rlo
            w0 = qq * PACK_W

            def _go():
                pltpu.sync_copy(yf_hbm.at[b, pl.ds(rlo, nr)], ybuf.at[pl.ds(0, nr)])

                def row_body(r, _):
                    i = rlo + r
                    base = 399 * i - jnp.right_shift(i * (i - 1), 1) - i - 1 - w0
                    v0 = jnp.right_shift(i + 1, 4)

                    def col_body(c, _):
                        j = c * 16 + iot
                        e = base + j
                        m = (j > jnp.broadcast_to(i, (16,))) & (e >= 0) & (e < PACK_W)
                        ec = jnp.clip(e, 0, PACK_W - 1)
                        vals = ybuf[r, pl.ds(c * 16, 16)]
                        plsc.store_scatter(obuf, [ec], vals, mask=m)
                        return 0

                    lax.fori_loop(v0, 25, col_body, 0)
                    return 0

                lax.fori_loop(0, nr, row_body, 0)
                pltpu.sync_copy(obuf, out_hbm.at[pl.ds(b * (4 * PACK_W) + w0, PACK_W)])

            return _go

        for qq in range(4):
            pl.when(q == qq)(variant(qq))

    return _sc_pack


def kernel(x, W1, b1, a1, W2, b2, a2, Wm1, bm1, am, Wm2, bm2, label_edge, mp_edge):
    del label_edge, mp_edge
    xaug = jnp.concatenate(
        [x, jnp.ones((BSZ, 1), jnp.float32),
         jnp.zeros((BSZ, XROWS * 128 - UT - 1), jnp.float32)],
        axis=1,
    ).reshape(BSZ * XROWS * 128)
    A = _sc_unpack_call()(xaug)
    scal = jnp.stack([a1, a2, am, bm2[0]]).reshape(1, 4)
    yfull = _tc_dense(
        A, W1.T, W2.T, Wm1[:, :CONV].T, Wm1[:, CONV:].T, Wm2,
        b1.reshape(1, CONV), b2.reshape(1, CONV), bm1.reshape(1, DNN), scal,
    )
    ypad = _sc_pack_call()(yfull)
    return ypad.reshape(BSZ, 4 * PACK_W)[:, :UT]


# speedup vs baseline: 326.5910x; 1.2300x over previous
import functools

import jax
import jax.numpy as jnp
from jax import lax
from jax.experimental import pallas as pl
from jax.experimental.pallas import tpu as pltpu
from jax.experimental.pallas import tpu_sc as plsc

N = 400
BSZ = 8
UT = N * (N - 1) // 2
CONV = 128
DNN = 64
XROWS = 624
UNPACK_CHUNKS = (
    ((0, 56), (56, 48)),
    ((104, 56), (160, 48)),
    ((208, 48), (256, 48)),
    ((304, 48), (352, 48)),
)
UNPACK_BUF_ROWS = 56
PACK_W = 20480
PACK_ROWS = ((0, 56), (48, 128), (120, 208), (200, 400))
PACK_MAX_ROWS = 200


def _iota16():
    return lax.broadcasted_iota(jnp.int32, (16,), 0)


@functools.cache
def _sc_mesh():
    return plsc.VectorSubcoreMesh(
        core_axis_name="c", subcore_axis_name="s", num_cores=2, num_subcores=16
    )


@functools.cache
def _sc_unpack_call():
    @functools.partial(
        pl.kernel,
        out_type=jax.ShapeDtypeStruct((BSZ, N, N), jnp.float32),
        mesh=_sc_mesh(),
        scratch_types=[
            pltpu.VMEM((XROWS * 128,), jnp.float32),
            pltpu.VMEM((UNPACK_BUF_ROWS, N), jnp.float32),
        ],
        compiler_params=pltpu.CompilerParams(needs_layout_passes=False),
    )
    def _sc_unpack(x_hbm, a_hbm, xbuf, obuf):
        wid = lax.axis_index("s") * 2 + lax.axis_index("c")
        b = wid // 4
        q = wid % 4
        pltpu.sync_copy(x_hbm.at[pl.ds(b * (XROWS * 128), XROWS * 128)], xbuf)
        iot = _iota16()

        def variant(qq):
            def _go():
                for r0, cnt in UNPACK_CHUNKS[qq]:
                    def row_body(r, _, r0=r0):
                        i = r0 + r

                        def col_body(c, _):
                            j = c * 16 + iot
                            iv = jnp.broadcast_to(i, (16,))
                            lo = jnp.minimum(iv, j)
                            hi = jnp.maximum(iv, j)
                            idx = (399 * lo - jnp.right_shift(lo * (lo - 1), 1)
                                   + hi - lo - 1)
                            idx = jnp.where(iv == j, UT, idx)
                            vals = plsc.load_gather(xbuf, [idx])
                            obuf[r, pl.ds(c * 16, 16)] = vals
                            return 0

                        lax.fori_loop(0, 25, col_body, 0)
                        return 0

                    lax.fori_loop(0, cnt, row_body, 0)
                    pltpu.sync_copy(
                        obuf.at[pl.ds(0, cnt)], a_hbm.at[b, pl.ds(r0, cnt)]
                    )

            return _go

        for qq in range(4):
            pl.when(q == qq)(variant(qq))

    return _sc_unpack


def _tc_body(a_ref, w1t_ref, w2t_ref, wma_ref, wmb_ref, wm2_ref,
             b1_ref, b2_ref, bm1_ref, scal_ref, y_ref):
    a1 = scal_ref[0, 0]
    a2 = scal_ref[0, 1]
    am = scal_ref[0, 2]
    bm2 = scal_ref[0, 3]

    A = a_ref[0]
    deg = jnp.sum(A, axis=1, keepdims=True)
    degr = jnp.sum(A, axis=0, keepdims=True)
    dc = jnp.where(deg > 0, lax.rsqrt(deg), 0.0)
    dr = jnp.where(degr > 0, lax.rsqrt(degr), 0.0)
    An = A * dc * dr

    t1 = jnp.dot(An, w1t_ref[...], preferred_element_type=jnp.float32) + b1_ref[...]
    h1 = jnp.where(t1 >= 0, t1, a1 * t1)
    t2a = jnp.dot(h1, w2t_ref[...], preferred_element_type=jnp.float32)
    t2 = jnp.dot(An, t2a, preferred_element_type=jnp.float32) + b2_ref[...]
    h2 = jnp.where(t2 >= 0, t2, a2 * t2)

    P = jnp.dot(h2, wma_ref[...], preferred_element_type=jnp.float32) + bm1_ref[...]
    Q = jnp.dot(h2, wmb_ref[...], preferred_element_type=jnp.float32)
    QT = Q.T

    c1 = (1.0 + am) * 0.5
    c2 = (1.0 - am) * 0.5
    wm2row = wm2_ref[...]
    Pw = lax.dot_general(P, wm2row, (((1,), (1,)), ((), ())),
                         preferred_element_type=jnp.float32)
    QwT = lax.dot_general(wm2row, QT, (((1,), (0,)), ((), ())),
                          preferred_element_type=jnp.float32)
    base = bm2 + c1 * (Pw + QwT)

    col_w = (128, 128, 128, N - 384)
    row_r = (128, 256, 384, N)
    accs = [jnp.zeros((row_r[jb], col_w[jb]), jnp.float32) for jb in range(4)]
    for k in range(DNN):
        wk = wm2_ref[0, k]
        pcol = P[:, k:k + 1]
        qrow = QT[k:k + 1, :]
        for jb in range(4):
            t = pcol[:row_r[jb]] + qrow[:, jb * 128:jb * 128 + col_w[jb]]
            accs[jb] = accs[jb] + wk * jnp.abs(t)
    for jb in range(4):
        sl = (slice(0, row_r[jb]), slice(jb * 128, jb * 128 + col_w[jb]))
        y_ref[0, sl[0], sl[1]] = base[sl[0], sl[1]] + c2 * accs[jb]


def _tc_dense(A, w1t, w2t, wma, wmb, wm2, b1r, b2r, bm1r, scal):
    full = lambda shape: pl.BlockSpec(shape, lambda b: (0,) * len(shape))
    return pl.pallas_call(
        _tc_body,
        grid=(BSZ,),
        in_specs=[
            pl.BlockSpec((1, N, N), lambda b: (b, 0, 0)),
            full((N, CONV)), full((CONV, CONV)), full((CONV, DNN)),
            full((CONV, DNN)), full((1, DNN)), full((1, CONV)),
            full((1, CONV)), full((1, DNN)), full((1, 4)),
        ],
        out_specs=pl.BlockSpec((1, N, N), lambda b: (b, 0, 0)),
        out_shape=jax.ShapeDtypeStruct((BSZ, N, N), jnp.float32),
    )(A, w1t, w2t, wma, wmb, wm2, b1r, b2r, bm1r, scal)


@functools.cache
def _sc_pack_call():
    @functools.partial(
        pl.kernel,
        out_type=jax.ShapeDtypeStruct((BSZ * 4 * PACK_W,), jnp.float32),
        mesh=_sc_mesh(),
        scratch_types=[
            pltpu.VMEM((PACK_MAX_ROWS, N), jnp.float32),
            pltpu.VMEM((PACK_W,), jnp.float32),
        ],
        compiler_params=pltpu.CompilerParams(needs_layout_passes=False),
    )
    def _sc_pack(yf_hbm, out_hbm, ybuf, obuf):
        wid = lax.axis_index("s") * 2 + lax.axis_index("c")
        b = wid // 4
        q = wid % 4
        iot = _iota16()

        def variant(qq):
            rlo, rhi = PACK_ROWS[qq]
            nr = rhi - rlo
            w0 = qq * PACK_W

            def _go():
                pltpu.sync_copy(yf_hbm.at[b, pl.ds(rlo, nr)], ybuf.at[pl.ds(0, nr)])

                def row_body(r, _):
                    i = rlo + r
                    base = 399 * i - jnp.right_shift(i * (i - 1), 1) - i - 1 - w0
                    v0 = jnp.right_shift(i + 1, 4)

                    def col_body(c, _):
                        j = c * 16 + iot
                        e = base + j
                        m = (j > jnp.broadcast_to(i, (16,))) & (e >= 0) & (e < PACK_W)
                        ec = jnp.clip(e, 0, PACK_W - 1)
                        vals = ybuf[r, pl.ds(c * 16, 16)]
                        plsc.store_scatter(obuf, [ec], vals, mask=m)
                        return 0

                    lax.fori_loop(v0, 25, col_body, 0)
                    return 0

                lax.fori_loop(0, nr, row_body, 0)
                pltpu.sync_copy(obuf, out_hbm.at[pl.ds(b * (4 * PACK_W) + w0, PACK_W)])

            return _go

        for qq in range(4):
            pl.when(q == qq)(variant(qq))

    return _sc_pack


def kernel(x, W1, b1, a1, W2, b2, a2, Wm1, bm1, am, Wm2, bm2, label_edge, mp_edge):
    del label_edge, mp_edge
    xaug = jnp.concatenate(
        [x, jnp.ones((BSZ, 1), jnp.float32),
         jnp.zeros((BSZ, XROWS * 128 - UT - 1), jnp.float32)],
        axis=1,
    ).reshape(BSZ * XROWS * 128)
    A = _sc_unpack_call()(xaug)
    scal = jnp.stack([a1, a2, am, bm2[0]]).reshape(1, 4)
    yfull = _tc_dense(
        A, W1.T, W2.T, Wm1[:, :CONV].T, Wm1[:, CONV:].T, Wm2,
        b1.reshape(1, CONV), b2.reshape(1, CONV), bm1.reshape(1, DNN), scal,
    )
    ypad = _sc_pack_call()(yfull)
    return ypad.reshape(BSZ, 4 * PACK_W)[:, :UT]
